# Initial kernel scaffold; baseline (speedup 1.0000x reference)
import jax
import jax.numpy as jnp
from jax.experimental import pallas as pl


def kernel(nodes, edges, segmentation_index, index, W_node, W_e1, b_e1, W_e2, b_e2):
    raise NotImplementedError("write your pallas kernel here")


# speedup vs baseline: 2.5074x; 2.5074x over previous
import functools

import jax
import jax.numpy as jnp
from jax import lax
from jax.experimental import pallas as pl
from jax.experimental.pallas import tpu as pltpu
from jax.experimental.pallas import tpu_sc as plsc

N_NODES = 10000
N_EDGES = 320000
D_NODE = 128
D_EDGE = 16
D_HID = 128

NC = 2
NS = 16
NW = NC * NS
E_PER_W = N_EDGES // NW
K = 80
CHUNKS = E_PER_W // K
N_PAD = 10240
ROWS_PER_TILE = N_PAD // NS


def _leaky(x):
    return jnp.where(x >= 0, x, 0.01 * x)


def _edge_proj_body(e_ref, w1_ref, b1_ref, w2_ref, b2_ref, o_ref):
    h = jnp.dot(e_ref[...], w1_ref[...], preferred_element_type=jnp.float32)
    h = _leaky(h + b1_ref[...])
    h = jnp.dot(h, w2_ref[...], preferred_element_type=jnp.float32)
    o_ref[...] = _leaky(h + b2_ref[...])


def _edge_proj(edges, w1, b1, w2, b2):
    BLK = 8000
    return pl.pallas_call(
        _edge_proj_body,
        grid=(N_EDGES // BLK,),
        in_specs=[
            pl.BlockSpec((BLK, D_EDGE), lambda i: (i, 0)),
            pl.BlockSpec((D_EDGE, D_HID), lambda i: (0, 0)),
            pl.BlockSpec((1, D_HID), lambda i: (0, 0)),
            pl.BlockSpec((D_HID, D_HID), lambda i: (0, 0)),
            pl.BlockSpec((1, D_HID), lambda i: (0, 0)),
        ],
        out_specs=pl.BlockSpec((BLK, D_HID), lambda i: (i, 0)),
        out_shape=jax.ShapeDtypeStruct((N_EDGES, D_HID), jnp.float32),
    )(edges, w1, b1.reshape(1, D_HID), w2, b2.reshape(1, D_HID))


def _node_proj_body(n_ref, w_ref, o_ref):
    o_ref[...] = jnp.dot(n_ref[...], w_ref[...],
                         preferred_element_type=jnp.float32)


def _node_proj(nodes, w):
    BLK = 2000
    return pl.pallas_call(
        _node_proj_body,
        grid=(N_NODES // BLK,),
        in_specs=[
            pl.BlockSpec((BLK, D_NODE), lambda i: (i, 0)),
            pl.BlockSpec((D_NODE, D_HID), lambda i: (0, 0)),
        ],
        out_specs=pl.BlockSpec((BLK, D_HID), lambda i: (i, 0)),
        out_shape=jax.ShapeDtypeStruct((N_NODES, D_HID), jnp.float32),
    )(nodes, w)


def _combine_body(a_ref, b_ref, o_ref):
    o_ref[...] = a_ref[...] + b_ref[...]


def _combine(a, b):
    BLK = 2000
    return pl.pallas_call(
        _combine_body,
        grid=(N_NODES // BLK,),
        in_specs=[
            pl.BlockSpec((BLK, D_NODE), lambda i: (i, 0)),
            pl.BlockSpec((BLK, D_NODE), lambda i: (i, 0)),
        ],
        out_specs=pl.BlockSpec((BLK, D_NODE), lambda i: (i, 0)),
        out_shape=jax.ShapeDtypeStruct((N_NODES, D_NODE), jnp.float32),
    )(a, b)


@functools.partial(
    pl.kernel,
    out_type=(
        jax.ShapeDtypeStruct((N_PAD, D_NODE), jnp.float32),
        jax.ShapeDtypeStruct((N_PAD, D_NODE), jnp.float32),
    ),
    mesh=plsc.VectorSubcoreMesh(core_axis_name="c", subcore_axis_name="s"),
    scratch_types=[
        pltpu.VMEM((K,), jnp.int32),
        pltpu.VMEM((K,), jnp.int32),
        pltpu.VMEM((K, D_NODE), jnp.float32),
        pltpu.VMEM((K, D_NODE), jnp.float32),
        pltpu.VMEM_SHARED((N_PAD, D_NODE), jnp.float32),
        pltpu.SemaphoreType.DMA,
    ],
)
def _sc_gather_scatter(m_hbm, e_hbm, idx_hbm, seg_hbm, z_hbm,
                       out0_hbm, out1_hbm,
                       idx_v, seg_v, rows_v, e_v, acc, sem):
    c = lax.axis_index("c")
    s = lax.axis_index("s")
    wid = s * NC + c

    pltpu.sync_copy(z_hbm, acc.at[pl.ds(s * ROWS_PER_TILE, ROWS_PER_TILE)])
    plsc.subcore_barrier()

    def chunk_body(i, carry):
        base = pl.multiple_of(wid * E_PER_W + i * K, 16)
        pltpu.sync_copy(idx_hbm.at[pl.ds(base, K)], idx_v)
        pltpu.sync_copy(seg_hbm.at[pl.ds(base, K)], seg_v)
        pltpu.async_copy(m_hbm.at[idx_v], rows_v, sem).wait()
        pltpu.sync_copy(e_hbm.at[pl.ds(base, K)], e_v)

        def row_body(r, carry2):
            for j in range(D_NODE // 16):
                sl = pl.ds(j * 16, 16)
                rows_v[r, sl] = rows_v[r, sl] * e_v[r, sl]
            return carry2

        lax.fori_loop(0, K, row_body, 0)
        pltpu.sync_copy(rows_v, acc.at[seg_v], add=True)
        return carry

    lax.fori_loop(0, CHUNKS, chunk_body, 0)
    plsc.subcore_barrier()

    row0 = s * ROWS_PER_TILE
    acc_slice = acc.at[pl.ds(row0, ROWS_PER_TILE)]

    @pl.when(c == 0)
    def _():
        pltpu.sync_copy(acc_slice, out0_hbm.at[pl.ds(row0, ROWS_PER_TILE)])

    @pl.when(c == 1)
    def _():
        pltpu.sync_copy(acc_slice, out1_hbm.at[pl.ds(row0, ROWS_PER_TILE)])


def kernel(nodes, edges, segmentation_index, index, W_node, W_e1, b_e1, W_e2,
           b_e2):
    idx = index.astype(jnp.int32)
    seg = segmentation_index.astype(jnp.int32)
    e = _edge_proj(edges, W_e1, b_e1, W_e2, b_e2)
    m = _node_proj(nodes, W_node)
    z = jnp.zeros((ROWS_PER_TILE, D_NODE), jnp.float32)
    p0, p1 = _sc_gather_scatter(m, e, idx, seg, z)
    return _combine(p0[:N_NODES], p1[:N_NODES])


# speedup vs baseline: 4.4415x; 1.7713x over previous
import functools

import jax
import jax.numpy as jnp
from jax import lax
from jax.experimental import pallas as pl
from jax.experimental.pallas import tpu as pltpu
from jax.experimental.pallas import tpu_sc as plsc

N_NODES = 10000
N_EDGES = 320000
D_NODE = 128
D_EDGE = 16
D_HID = 128

NC = 2
NS = 16
NW = NC * NS
E_PER_W = N_EDGES // NW
K = 80
CHUNKS = E_PER_W // K
N_PAD = 10240
ROWS_PER_TILE = N_PAD // NS


def _leaky(x):
    return jnp.where(x >= 0, x, 0.01 * x)


def _edge_proj_body(e_ref, w1_ref, b1_ref, w2_ref, b2_ref, o_ref):
    h = jnp.dot(e_ref[...], w1_ref[...], preferred_element_type=jnp.float32)
    h = _leaky(h + b1_ref[...])
    h = jnp.dot(h, w2_ref[...], preferred_element_type=jnp.float32)
    o_ref[...] = _leaky(h + b2_ref[...])


def _edge_proj(edges, w1, b1, w2, b2):
    BLK = 8000
    return pl.pallas_call(
        _edge_proj_body,
        grid=(N_EDGES // BLK,),
        in_specs=[
            pl.BlockSpec((BLK, D_EDGE), lambda i: (i, 0)),
            pl.BlockSpec((D_EDGE, D_HID), lambda i: (0, 0)),
            pl.BlockSpec((1, D_HID), lambda i: (0, 0)),
            pl.BlockSpec((D_HID, D_HID), lambda i: (0, 0)),
            pl.BlockSpec((1, D_HID), lambda i: (0, 0)),
        ],
        out_specs=pl.BlockSpec((BLK, D_HID), lambda i: (i, 0)),
        out_shape=jax.ShapeDtypeStruct((N_EDGES, D_HID), jnp.float32),
    )(edges, w1, b1.reshape(1, D_HID), w2, b2.reshape(1, D_HID))


def _node_proj_body(n_ref, w_ref, o_ref):
    o_ref[...] = jnp.dot(n_ref[...], w_ref[...],
                         preferred_element_type=jnp.float32)


def _node_proj(nodes, w):
    BLK = 2000
    return pl.pallas_call(
        _node_proj_body,
        grid=(N_NODES // BLK,),
        in_specs=[
            pl.BlockSpec((BLK, D_NODE), lambda i: (i, 0)),
            pl.BlockSpec((D_NODE, D_HID), lambda i: (0, 0)),
        ],
        out_specs=pl.BlockSpec((BLK, D_HID), lambda i: (i, 0)),
        out_shape=jax.ShapeDtypeStruct((N_NODES, D_HID), jnp.float32),
    )(nodes, w)


def _combine_body(a_ref, b_ref, o_ref):
    o_ref[...] = a_ref[...] + b_ref[...]


def _combine(a, b):
    BLK = 2000
    return pl.pallas_call(
        _combine_body,
        grid=(N_NODES // BLK,),
        in_specs=[
            pl.BlockSpec((BLK, D_NODE), lambda i: (i, 0)),
            pl.BlockSpec((BLK, D_NODE), lambda i: (i, 0)),
        ],
        out_specs=pl.BlockSpec((BLK, D_NODE), lambda i: (i, 0)),
        out_shape=jax.ShapeDtypeStruct((N_NODES, D_NODE), jnp.float32),
    )(a, b)


@functools.partial(
    pl.kernel,
    out_type=(
        jax.ShapeDtypeStruct((N_PAD, D_NODE), jnp.float32),
        jax.ShapeDtypeStruct((N_PAD, D_NODE), jnp.float32),
    ),
    mesh=plsc.VectorSubcoreMesh(core_axis_name="c", subcore_axis_name="s"),
    scratch_types=[
        pltpu.VMEM((K,), jnp.int32),
        pltpu.VMEM((K,), jnp.int32),
        pltpu.VMEM((K,), jnp.int32),
        pltpu.VMEM((K,), jnp.int32),
        pltpu.VMEM((K,), jnp.int32),
        pltpu.VMEM((K,), jnp.int32),
        pltpu.VMEM((K,), jnp.int32),
        pltpu.VMEM((K,), jnp.int32),
        pltpu.VMEM((2, K, D_NODE), jnp.float32),
        pltpu.VMEM((2, K, D_NODE), jnp.float32),
        pltpu.VMEM_SHARED((N_PAD, D_NODE), jnp.float32),
        pltpu.SemaphoreType.DMA,
        pltpu.SemaphoreType.DMA,
        pltpu.SemaphoreType.DMA,
        pltpu.SemaphoreType.DMA,
        pltpu.SemaphoreType.DMA,
        pltpu.SemaphoreType.DMA,
        pltpu.SemaphoreType.DMA,
        pltpu.SemaphoreType.DMA,
    ],
)
def _sc_gather_scatter(m_hbm, e_hbm, idx_hbm, seg_hbm, z_hbm,
                       out0_hbm, out1_hbm,
                       idx_r0, idx_r1, idx_r2, idx_r3,
                       seg_r0, seg_r1, seg_r2, seg_r3,
                       rows_v, e_v, acc,
                       gsem0, gsem1, esem0, esem1,
                       isem0, isem1, isem2, isem3):
    c = lax.axis_index("c")
    s = lax.axis_index("s")
    wid = s * NC + c
    gsems = (gsem0, gsem1)
    esems = (esem0, esem1)
    isems = (isem0, isem1, isem2, isem3)
    idx_r = (idx_r0, idx_r1, idx_r2, idx_r3)
    seg_r = (seg_r0, seg_r1, seg_r2, seg_r3)

    pltpu.sync_copy(z_hbm, acc.at[pl.ds(s * ROWS_PER_TILE, ROWS_PER_TILE)])

    def _idx_src(ci):
        base = pl.multiple_of(wid * E_PER_W + ci * K, 16)
        return idx_hbm.at[pl.ds(base, K)], seg_hbm.at[pl.ds(base, K)]

    def _e_src(ci):
        base = pl.multiple_of(wid * E_PER_W + ci * K, 16)
        return e_hbm.at[pl.ds(base, K)]

    def _issue_idx(ci, slot):
        isrc, ssrc = _idx_src(ci)
        pltpu.async_copy(isrc, idx_r[slot], isems[slot])
        pltpu.async_copy(ssrc, seg_r[slot], isems[slot])

    def _wait_idx(ci, slot):
        isrc, ssrc = _idx_src(ci)
        pltpu.make_async_copy(isrc, idx_r[slot], isems[slot]).wait()
        pltpu.make_async_copy(ssrc, seg_r[slot], isems[slot]).wait()

    def _issue_data(ci, slot, b):
        pltpu.async_copy(m_hbm.at[idx_r[slot]], rows_v.at[b], gsems[b])
        pltpu.async_copy(_e_src(ci), e_v.at[b], esems[b])

    def _wait_data(ci, slot, b):
        pltpu.make_async_copy(m_hbm.at[idx_r[slot]], rows_v.at[b],
                              gsems[b]).wait()
        pltpu.make_async_copy(_e_src(ci), e_v.at[b], esems[b]).wait()

    def _process(ci, slot, b):
        _wait_data(ci, slot, b)

        def row_body(r, carry2):
            for j in range(D_NODE // 16):
                sl = pl.ds(j * 16, 16)
                rows_v[b, r, sl] = rows_v[b, r, sl] * e_v[b, r, sl]
            return carry2

        lax.fori_loop(0, K, row_body, 0)
        pltpu.sync_copy(rows_v.at[b], acc.at[seg_r[slot]], add=True)

    for ci in range(4):
        _issue_idx(ci, ci)
    for ci in range(2):
        _wait_idx(ci, ci)
        _issue_data(ci, ci, ci)
    plsc.subcore_barrier()

    @pl.loop(0, CHUNKS - 1, step=4)
    def _main(i):
        for b in range(4):
            ci = i + b
            _process(ci, b, b % 2)

            @pl.when(ci + 2 < CHUNKS)
            def _():
                _wait_idx(ci + 2, (b + 2) % 4)
                _issue_data(ci + 2, (b + 2) % 4, b % 2)

            @pl.when(ci + 4 < CHUNKS)
            def _():
                _issue_idx(ci + 4, b)

    _process(CHUNKS - 1, 0, 0)
    plsc.subcore_barrier()

    row0 = s * ROWS_PER_TILE
    acc_slice = acc.at[pl.ds(row0, ROWS_PER_TILE)]

    @pl.when(c == 0)
    def _():
        pltpu.sync_copy(acc_slice, out0_hbm.at[pl.ds(row0, ROWS_PER_TILE)])

    @pl.when(c == 1)
    def _():
        pltpu.sync_copy(acc_slice, out1_hbm.at[pl.ds(row0, ROWS_PER_TILE)])


def kernel(nodes, edges, segmentation_index, index, W_node, W_e1, b_e1, W_e2,
           b_e2):
    idx = index.astype(jnp.int32)
    seg = segmentation_index.astype(jnp.int32)
    e = _edge_proj(edges, W_e1, b_e1, W_e2, b_e2)
    m = _node_proj(nodes, W_node)
    z = jnp.zeros((ROWS_PER_TILE, D_NODE), jnp.float32)
    p0, p1 = _sc_gather_scatter(m, e, idx, seg, z)
    return _combine(p0[:N_NODES], p1[:N_NODES])
